# Initial kernel scaffold; baseline (speedup 1.0000x reference)
import jax
import jax.numpy as jnp
from jax.experimental import pallas as pl


def kernel(x, edge_index, W1, b1, W2, b2, W3, b3, Wo, bo):
    raise NotImplementedError("write your pallas kernel here")


# speedup vs baseline: 3.8697x; 3.8697x over previous
import functools

import jax
import jax.numpy as jnp
from jax import lax
from jax.experimental import pallas as pl
from jax.experimental.pallas import tpu as pltpu
from jax.experimental.pallas import tpu_sc as plsc

N = 10000
E = 160000
D_IN = 256
D_H = 512
D_OUT = 128

K = 128
NB = 40
E_PAD = 2 * 16 * NB * K
PAD_DST = 10008
NPAD = 10240
ZR = 40
OUT_STEP = 624
OUT_ROWS = 640

BM = 2000
GM = N // BM


def _scatter_sc(C, use_ones):
  mesh = plsc.VectorSubcoreMesh(core_axis_name="c", subcore_axis_name="s")

  @functools.partial(
      pl.kernel,
      mesh=mesh,
      out_type=jax.ShapeDtypeStruct((2, C, N, 128), jnp.float32),
      scratch_types=[
          pltpu.VMEM_SHARED((NPAD, 128), jnp.float32),
          pltpu.VMEM((ZR, 128), jnp.float32),
          pltpu.VMEM((NB, K), jnp.int32),
          pltpu.VMEM((NB, K), jnp.int32),
          pltpu.VMEM((K, 128), jnp.float32),
          pltpu.VMEM((K, 128), jnp.float32),
          pltpu.SemaphoreType.DMA,
          pltpu.SemaphoreType.DMA,
      ],
  )
  def scatter(g_hbm, src_hbm, dst_hbm, zeros_hbm, ones_hbm, out_hbm,
              acc, zbuf, src_v, dst_v, rows_a, rows_b, sem_a, sem_b):
    core = lax.axis_index("c")
    sid = lax.axis_index("s")

    pltpu.sync_copy(dst_hbm.at[core, sid], dst_v)
    pltpu.sync_copy(zeros_hbm, zbuf)
    if use_ones:
      pltpu.sync_copy(ones_hbm, rows_a)
    else:
      pltpu.sync_copy(src_hbm.at[core, sid], src_v)

    for c in range(C):
      if c > 0:
        def shift_row(j, _):
          def shift16(l, _):
            sl = pl.ds(l * 16, 16)
            src_v[j, sl] = src_v[j, sl] + jnp.full((16,), N, jnp.int32)
            return 0
          lax.fori_loop(0, K // 16, shift16, 0)
          return 0
        lax.fori_loop(0, NB, shift_row, 0)
      def zinit(q, _):
        pltpu.sync_copy(zbuf, acc.at[pl.ds(sid * 16 * ZR + q * ZR, ZR)])
        return 0
      lax.fori_loop(0, 16, zinit, 0)
      plsc.subcore_barrier()

      if use_ones:
        def body1(j, _):
          pltpu.sync_copy(rows_a, acc.at[dst_v.at[j]], add=True)
          return 0
        lax.fori_loop(0, NB, body1, 0)
      else:
        pltpu.async_copy(g_hbm.at[src_v.at[0]], rows_a, sem_a)

        def body2(i, _):
          ja = 2 * i
          pltpu.async_copy(g_hbm.at[src_v.at[ja + 1]], rows_b, sem_b)
          pltpu.make_async_copy(g_hbm.at[src_v.at[ja]], rows_a, sem_a).wait()
          pltpu.sync_copy(rows_a, acc.at[dst_v.at[ja]], add=True)

          @pl.when(ja + 2 < NB)
          def _():
            pltpu.async_copy(g_hbm.at[src_v.at[ja + 2]], rows_a, sem_a)

          pltpu.make_async_copy(g_hbm.at[src_v.at[ja + 1]], rows_b, sem_b).wait()
          pltpu.sync_copy(rows_b, acc.at[dst_v.at[ja + 1]], add=True)
          return 0

        lax.fori_loop(0, NB // 2, body2, 0)

      plsc.subcore_barrier()
      pltpu.sync_copy(acc.at[pl.ds(sid * OUT_STEP, OUT_ROWS)],
                      out_hbm.at[core, c, pl.ds(sid * OUT_STEP, OUT_ROWS)])
      plsc.subcore_barrier()

  return scatter


def _dinv_body(d0_ref, d1_ref, out_ref):
  out_ref[...] = lax.rsqrt(d0_ref[...] + d1_ref[...] + 1.0)


def _dinv_call(d0, d1):
  return pl.pallas_call(
      _dinv_body,
      grid=(GM,),
      in_specs=[pl.BlockSpec((BM, 128), lambda i: (i, 0)),
                pl.BlockSpec((BM, 128), lambda i: (i, 0))],
      out_specs=pl.BlockSpec((BM, 128), lambda i: (i, 0)),
      out_shape=jax.ShapeDtypeStruct((N, 128), jnp.float32),
  )(d0, d1)


def _k1_body(x_ref, w_ref, dinv_ref, out_ref):
  out_ref[...] = jnp.dot(x_ref[...], w_ref[...],
                         preferred_element_type=jnp.float32) * dinv_ref[...]


def _k1_call(x, W1, dinvp):
  co = D_H // 128
  return pl.pallas_call(
      _k1_body,
      grid=(co, GM),
      in_specs=[
          pl.BlockSpec((BM, D_IN), lambda c, i: (i, 0)),
          pl.BlockSpec((D_IN, 128), lambda c, i: (0, c)),
          pl.BlockSpec((BM, 128), lambda c, i: (i, 0)),
      ],
      out_specs=pl.BlockSpec((BM, 128), lambda c, i: (c * GM + i, 0)),
      out_shape=jax.ShapeDtypeStruct((co * N, 128), jnp.float32),
      compiler_params=pltpu.CompilerParams(
          dimension_semantics=("parallel", "parallel")),
  )(x, W1, dinvp)


def _kmid_body(t0_ref, t1_ref, g_ref, dinv_ref, b_ref, w_ref, out_ref, *,
               n_ci, scale_out):
  ci = pl.program_id(2)
  act = (t0_ref[0, 0] + t1_ref[0, 0] + g_ref[...]) * dinv_ref[...] + b_ref[0]
  act = jnp.maximum(act, 0.0)
  part = jnp.dot(act, w_ref[...], preferred_element_type=jnp.float32)

  @pl.when(ci == 0)
  def _():
    out_ref[...] = part

  @pl.when(ci > 0)
  def _():
    out_ref[...] += part

  if scale_out:
    @pl.when(ci == n_ci - 1)
    def _():
      out_ref[...] *= dinv_ref[...]


def _kmid_call(t, g, dinvp, b, W, d_out, scale_out=True):
  n_ci = D_H // 128
  n_co = d_out // 128
  body = functools.partial(_kmid_body, n_ci=n_ci, scale_out=scale_out)
  return pl.pallas_call(
      body,
      grid=(n_co, GM, n_ci),
      in_specs=[
          pl.BlockSpec((1, 1, BM, 128), lambda co, i, ci: (0, ci, i, 0)),
          pl.BlockSpec((1, 1, BM, 128), lambda co, i, ci: (1, ci, i, 0)),
          pl.BlockSpec((BM, 128), lambda co, i, ci: (ci * GM + i, 0)),
          pl.BlockSpec((BM, 128), lambda co, i, ci: (i, 0)),
          pl.BlockSpec((1, 1, 128), lambda co, i, ci: (ci, 0, 0)),
          pl.BlockSpec((128, 128), lambda co, i, ci: (ci, co)),
      ],
      out_specs=pl.BlockSpec((BM, 128), lambda co, i, ci: (co * GM + i, 0)),
      out_shape=jax.ShapeDtypeStruct((n_co * N, 128), jnp.float32),
      compiler_params=pltpu.CompilerParams(
          dimension_semantics=("parallel", "parallel", "arbitrary")),
  )(t, t, g, dinvp, b, W)


def _kfin_body(t0_ref, t1_ref, g_ref, dinv_ref, b_ref, out_ref):
  z = (t0_ref[0, 0] + t1_ref[0, 0] + g_ref[...]) * dinv_ref[...] + b_ref[0]
  m = jnp.max(z, axis=1, keepdims=True)
  e = jnp.exp(z - m)
  s = jnp.sum(e, axis=1, keepdims=True)
  out_ref[...] = z - m - jnp.log(s)


def _kfin_call(t, g, dinvp, bo):
  return pl.pallas_call(
      _kfin_body,
      grid=(GM,),
      in_specs=[
          pl.BlockSpec((1, 1, BM, 128), lambda i: (0, 0, i, 0)),
          pl.BlockSpec((1, 1, BM, 128), lambda i: (1, 0, i, 0)),
          pl.BlockSpec((BM, 128), lambda i: (i, 0)),
          pl.BlockSpec((BM, 128), lambda i: (i, 0)),
          pl.BlockSpec((1, 1, 128), lambda i: (0, 0, 0)),
      ],
      out_specs=pl.BlockSpec((BM, 128), lambda i: (i, 0)),
      out_shape=jax.ShapeDtypeStruct((N, D_OUT), jnp.float32),
  )(t, t, g, dinvp, bo)


def kernel(x, edge_index, W1, b1, W2, b2, W3, b3, Wo, bo):
  src = edge_index[0].astype(jnp.int32)
  dst = edge_index[1].astype(jnp.int32)

  npad = E_PAD - E
  src_p = jnp.concatenate([src, jnp.zeros((npad,), jnp.int32)])
  dst_p = jnp.concatenate([dst, jnp.full((npad,), PAD_DST, jnp.int32)])
  dst_t = dst_p.reshape(2, 16, NB, K)
  src_t = src_p.reshape(2, 16, NB, K)

  zeros_st = jnp.zeros((ZR, 128), jnp.float32)
  ones_st = jnp.ones((K, 128), jnp.float32)
  dummy_g = jnp.zeros((8, 128), jnp.float32)

  degt = _scatter_sc(1, True)(dummy_g, src_t, dst_t, zeros_st, ones_st)
  dinvp = _dinv_call(degt[0, 0], degt[1, 0])

  scat4 = _scatter_sc(4, False)
  scat1 = _scatter_sc(1, False)

  b1r = b1.reshape(4, 1, 128)
  b2r = b2.reshape(4, 1, 128)
  b3r = b3.reshape(4, 1, 128)
  bor = bo.reshape(1, 1, 128)

  g1 = _k1_call(x, W1, dinvp)
  t1 = scat4(g1, src_t, dst_t, zeros_st, ones_st)
  g2 = _kmid_call(t1, g1, dinvp, b1r, W2, D_H)
  t2 = scat4(g2, src_t, dst_t, zeros_st, ones_st)
  g3 = _kmid_call(t2, g2, dinvp, b2r, W3, D_H)
  t3 = scat4(g3, src_t, dst_t, zeros_st, ones_st)
  g4 = _kmid_call(t3, g3, dinvp, b3r, Wo, D_OUT)
  t4 = scat1(g4, src_t, dst_t, zeros_st, ones_st)
  return _kfin_call(t4, g4, dinvp, bor)


# speedup vs baseline: 11.3648x; 2.9369x over previous
import functools

import jax
import jax.numpy as jnp
from jax import lax
from jax.experimental import pallas as pl
from jax.experimental.pallas import tpu as pltpu
from jax.experimental.pallas import tpu_sc as plsc

N = 10000
E = 160000
D_IN = 256
D_H = 512
D_OUT = 128

K = 128
NB = 40
E_PAD = 2 * 16 * NB * K
PAD_DST = 10008
NPAD = 10240
ZR = 40
OUT_STEP = 624
OUT_ROWS = 640

BM = 2000
GM = N // BM


def _scatter_sc(C, use_ones):
  mesh = plsc.VectorSubcoreMesh(core_axis_name="c", subcore_axis_name="s")

  @functools.partial(
      pl.kernel,
      mesh=mesh,
      out_type=jax.ShapeDtypeStruct((2, C, N, 128), jnp.float32),
      scratch_types=[
          pltpu.VMEM_SHARED((NPAD, 128), jnp.float32),
          pltpu.VMEM((ZR, 128), jnp.float32),
          pltpu.VMEM((NB, K), jnp.int32),
          pltpu.VMEM((NB, K), jnp.int32),
          pltpu.VMEM((K, 128), jnp.float32),
          pltpu.VMEM((K, 128), jnp.float32),
          pltpu.SemaphoreType.DMA,
          pltpu.SemaphoreType.DMA,
      ],
  )
  def scatter(g_hbm, src_hbm, dst_hbm, zeros_hbm, ones_hbm, out_hbm,
              acc, zbuf, src_v, dst_v, rows_a, rows_b, sem_a, sem_b):
    core = lax.axis_index("c")
    sid = lax.axis_index("s")

    pltpu.sync_copy(dst_hbm.at[core, sid], dst_v)
    pltpu.sync_copy(zeros_hbm, zbuf)
    if use_ones:
      pltpu.sync_copy(ones_hbm, rows_a)
    else:
      pltpu.sync_copy(src_hbm.at[core, sid], src_v)

    for c in range(C):
      if c > 0:
        def shift_row(j, _):
          def shift16(l, _):
            sl = pl.ds(l * 16, 16)
            src_v[j, sl] = src_v[j, sl] + jnp.full((16,), N, jnp.int32)
            return 0
          lax.fori_loop(0, K // 16, shift16, 0)
          return 0
        lax.fori_loop(0, NB, shift_row, 0)
      def zinit(q, _):
        pltpu.sync_copy(zbuf, acc.at[pl.ds(sid * 16 * ZR + q * ZR, ZR)])
        return 0
      lax.fori_loop(0, 16, zinit, 0)
      plsc.subcore_barrier()

      if use_ones:
        def body1(j, _):
          pltpu.sync_copy(rows_a, acc.at[dst_v.at[j]], add=True)
          return 0
        lax.fori_loop(0, NB, body1, 0)
      else:
        pltpu.async_copy(g_hbm.at[src_v.at[0]], rows_a, sem_a)

        def body2(i, _):
          ja = 2 * i
          pltpu.async_copy(g_hbm.at[src_v.at[ja + 1]], rows_b, sem_b)
          pltpu.make_async_copy(g_hbm.at[src_v.at[ja]], rows_a, sem_a).wait()
          pltpu.sync_copy(rows_a, acc.at[dst_v.at[ja]], add=True)

          @pl.when(ja + 2 < NB)
          def _():
            pltpu.async_copy(g_hbm.at[src_v.at[ja + 2]], rows_a, sem_a)

          pltpu.make_async_copy(g_hbm.at[src_v.at[ja + 1]], rows_b, sem_b).wait()
          pltpu.sync_copy(rows_b, acc.at[dst_v.at[ja + 1]], add=True)
          return 0

        lax.fori_loop(0, NB // 2, body2, 0)

      plsc.subcore_barrier()
      pltpu.sync_copy(acc.at[pl.ds(sid * OUT_STEP, OUT_ROWS)],
                      out_hbm.at[core, c, pl.ds(sid * OUT_STEP, OUT_ROWS)])
      plsc.subcore_barrier()

  return scatter


def _dinv_body(d0_ref, d1_ref, out_ref):
  out_ref[...] = lax.rsqrt(d0_ref[...] + d1_ref[...] + 1.0)


def _dinv_call(d0, d1):
  return pl.pallas_call(
      _dinv_body,
      grid=(GM,),
      in_specs=[pl.BlockSpec((BM, 128), lambda i: (i, 0)),
                pl.BlockSpec((BM, 128), lambda i: (i, 0))],
      out_specs=pl.BlockSpec((BM, 128), lambda i: (i, 0)),
      out_shape=jax.ShapeDtypeStruct((N, 128), jnp.float32),
  )(d0, d1)


def _k1_body(x_ref, w_ref, dinv_ref, out_ref):
  out_ref[...] = jnp.dot(x_ref[...], w_ref[...],
                         preferred_element_type=jnp.float32) * dinv_ref[...]


def _k1_call(x, W1, dinvp):
  co = D_H // 128
  return pl.pallas_call(
      _k1_body,
      grid=(co, GM),
      in_specs=[
          pl.BlockSpec((BM, D_IN), lambda c, i: (i, 0)),
          pl.BlockSpec((D_IN, 128), lambda c, i: (0, c)),
          pl.BlockSpec((BM, 128), lambda c, i: (i, 0)),
      ],
      out_specs=pl.BlockSpec((BM, 128), lambda c, i: (c * GM + i, 0)),
      out_shape=jax.ShapeDtypeStruct((co * N, 128), jnp.float32),
      compiler_params=pltpu.CompilerParams(
          dimension_semantics=("parallel", "parallel")),
  )(x, W1, dinvp)


def _kmid_body(t0_ref, t1_ref, g_ref, dinv_ref, b_ref, w_ref, out_ref, *,
               n_ci, scale_out):
  ci = pl.program_id(2)
  act = (t0_ref[0, 0] + t1_ref[0, 0] + g_ref[...]) * dinv_ref[...] + b_ref[0]
  act = jnp.maximum(act, 0.0)
  part = jnp.dot(act, w_ref[...], preferred_element_type=jnp.float32)

  @pl.when(ci == 0)
  def _():
    out_ref[...] = part

  @pl.when(ci > 0)
  def _():
    out_ref[...] += part

  if scale_out:
    @pl.when(ci == n_ci - 1)
    def _():
      out_ref[...] *= dinv_ref[...]


def _kmid_call(t, g, dinvp, b, W, d_out, scale_out=True):
  n_ci = D_H // 128
  n_co = d_out // 128
  body = functools.partial(_kmid_body, n_ci=n_ci, scale_out=scale_out)
  return pl.pallas_call(
      body,
      grid=(n_co, GM, n_ci),
      in_specs=[
          pl.BlockSpec((1, 1, BM, 128), lambda co, i, ci: (0, ci, i, 0)),
          pl.BlockSpec((1, 1, BM, 128), lambda co, i, ci: (1, ci, i, 0)),
          pl.BlockSpec((BM, 128), lambda co, i, ci: (ci * GM + i, 0)),
          pl.BlockSpec((BM, 128), lambda co, i, ci: (i, 0)),
          pl.BlockSpec((1, 1, 128), lambda co, i, ci: (ci, 0, 0)),
          pl.BlockSpec((128, 128), lambda co, i, ci: (ci, co)),
      ],
      out_specs=pl.BlockSpec((BM, 128), lambda co, i, ci: (co * GM + i, 0)),
      out_shape=jax.ShapeDtypeStruct((n_co * N, 128), jnp.float32),
      compiler_params=pltpu.CompilerParams(
          dimension_semantics=("parallel", "parallel", "arbitrary")),
  )(t, t, g, dinvp, b, W)


def _kfin_body(t0_ref, t1_ref, g_ref, dinv_ref, b_ref, out_ref):
  z = (t0_ref[0, 0] + t1_ref[0, 0] + g_ref[...]) * dinv_ref[...] + b_ref[0]
  m = jnp.max(z, axis=1, keepdims=True)
  e = jnp.exp(z - m)
  s = jnp.sum(e, axis=1, keepdims=True)
  out_ref[...] = z - m - jnp.log(s)


def _kfin_call(t, g, dinvp, bo):
  return pl.pallas_call(
      _kfin_body,
      grid=(GM,),
      in_specs=[
          pl.BlockSpec((1, 1, BM, 128), lambda i: (0, 0, i, 0)),
          pl.BlockSpec((1, 1, BM, 128), lambda i: (1, 0, i, 0)),
          pl.BlockSpec((BM, 128), lambda i: (i, 0)),
          pl.BlockSpec((BM, 128), lambda i: (i, 0)),
          pl.BlockSpec((1, 1, 128), lambda i: (0, 0, 0)),
      ],
      out_specs=pl.BlockSpec((BM, 128), lambda i: (i, 0)),
      out_shape=jax.ShapeDtypeStruct((N, D_OUT), jnp.float32),
  )(t, t, g, dinvp, bo)


def kernel(x, edge_index, W1, b1, W2, b2, W3, b3, Wo, bo):
  src = edge_index[0].astype(jnp.int32)
  dst = edge_index[1].astype(jnp.int32)

  npad = E_PAD - E
  pad_src = (jnp.arange(npad, dtype=jnp.int32) * 16) % N
  pad_dst = N + (jnp.arange(npad, dtype=jnp.int32) % (NPAD - N - 8))
  src_p = jnp.concatenate([src, pad_src])
  dst_p = jnp.concatenate([dst, pad_dst])
  dst_t = dst_p.reshape(2, 16, NB, K)
  src_t = src_p.reshape(2, 16, NB, K)

  zeros_st = jnp.zeros((ZR, 128), jnp.float32)
  ones_st = jnp.ones((K, 128), jnp.float32)
  dummy_g = jnp.zeros((8, 128), jnp.float32)

  degt = _scatter_sc(1, True)(dummy_g, src_t, dst_t, zeros_st, ones_st)
  dinvp = _dinv_call(degt[0, 0], degt[1, 0])

  scat4 = _scatter_sc(4, False)
  scat1 = _scatter_sc(1, False)

  b1r = b1.reshape(4, 1, 128)
  b2r = b2.reshape(4, 1, 128)
  b3r = b3.reshape(4, 1, 128)
  bor = bo.reshape(1, 1, 128)

  g1 = _k1_call(x, W1, dinvp)
  t1 = scat4(g1, src_t, dst_t, zeros_st, ones_st)
  g2 = _kmid_call(t1, g1, dinvp, b1r, W2, D_H)
  t2 = scat4(g2, src_t, dst_t, zeros_st, ones_st)
  g3 = _kmid_call(t2, g2, dinvp, b2r, W3, D_H)
  t3 = scat4(g3, src_t, dst_t, zeros_st, ones_st)
  g4 = _kmid_call(t3, g3, dinvp, b3r, Wo, D_OUT)
  t4 = scat1(g4, src_t, dst_t, zeros_st, ones_st)
  return _kfin_call(t4, g4, dinvp, bor)


# speedup vs baseline: 11.3844x; 1.0017x over previous
import functools

import jax
import jax.numpy as jnp
from jax import lax
from jax.experimental import pallas as pl
from jax.experimental.pallas import tpu as pltpu
from jax.experimental.pallas import tpu_sc as plsc

N = 10000
E = 160000
D_IN = 256
D_H = 512
D_OUT = 128

K = 128
NB = 40
E_PAD = 2 * 16 * NB * K
PAD_DST = 10008
NPAD = 10240
ZR = 40
OUT_STEP = 624
OUT_ROWS = 640

BM = 2000
GM = N // BM


def _scatter_sc(C, use_ones):
  mesh = plsc.VectorSubcoreMesh(core_axis_name="c", subcore_axis_name="s")

  @functools.partial(
      pl.kernel,
      mesh=mesh,
      out_type=jax.ShapeDtypeStruct((2, C, N, 128), jnp.float32),
      scratch_types=[
          pltpu.VMEM_SHARED((NPAD, 128), jnp.float32),
          pltpu.VMEM((ZR, 128), jnp.float32),
          pltpu.VMEM((NB, K), jnp.int32),
          pltpu.VMEM((NB, K), jnp.int32),
          pltpu.VMEM((K, 128), jnp.float32),
          pltpu.VMEM((K, 128), jnp.float32),
          pltpu.SemaphoreType.DMA,
          pltpu.SemaphoreType.DMA,
          pltpu.SemaphoreType.DMA,
          pltpu.SemaphoreType.DMA,
      ],
  )
  def scatter(g_hbm, src_hbm, dst_hbm, zeros_hbm, ones_hbm, out_hbm,
              acc, zbuf, src_v, dst_v, rows_a, rows_b,
              sem_a, sem_b, sem_sa, sem_sb):
    core = lax.axis_index("c")
    sid = lax.axis_index("s")

    pltpu.sync_copy(dst_hbm.at[core, sid], dst_v)
    pltpu.sync_copy(zeros_hbm, zbuf)
    if use_ones:
      pltpu.sync_copy(ones_hbm, rows_a)
    else:
      pltpu.sync_copy(src_hbm.at[core, sid], src_v)

    for c in range(C):
      if c > 0:
        def shift_row(j, _):
          def shift16(l, _):
            sl = pl.ds(l * 16, 16)
            src_v[j, sl] = src_v[j, sl] + jnp.full((16,), N, jnp.int32)
            return 0
          lax.fori_loop(0, K // 16, shift16, 0)
          return 0
        lax.fori_loop(0, NB, shift_row, 0)
      def zinit(q, _):
        pltpu.sync_copy(zbuf, acc.at[pl.ds(sid * 16 * ZR + q * ZR, ZR)])
        return 0
      lax.fori_loop(0, 16, zinit, 0)
      plsc.subcore_barrier()

      if use_ones:
        def body1(j, _):
          pltpu.sync_copy(rows_a, acc.at[dst_v.at[j]], add=True)
          return 0
        lax.fori_loop(0, NB, body1, 0)
      else:
        pltpu.async_copy(g_hbm.at[src_v.at[0]], rows_a, sem_a)

        def body2(i, _):
          ja = 2 * i

          @pl.when(i > 0)
          def _():
            pltpu.make_async_copy(rows_b, acc.at[dst_v.at[0]], sem_sb).wait()

          pltpu.async_copy(g_hbm.at[src_v.at[ja + 1]], rows_b, sem_b)
          pltpu.make_async_copy(g_hbm.at[src_v.at[ja]], rows_a, sem_a).wait()
          pltpu.async_copy(rows_a, acc.at[dst_v.at[ja]], sem_sa, add=True)

          @pl.when(ja + 2 < NB)
          def _():
            pltpu.make_async_copy(rows_a, acc.at[dst_v.at[0]], sem_sa).wait()
            pltpu.async_copy(g_hbm.at[src_v.at[ja + 2]], rows_a, sem_a)

          pltpu.make_async_copy(g_hbm.at[src_v.at[ja + 1]], rows_b, sem_b).wait()
          pltpu.async_copy(rows_b, acc.at[dst_v.at[ja + 1]], sem_sb, add=True)
          return 0

        lax.fori_loop(0, NB // 2, body2, 0)
        pltpu.make_async_copy(rows_a, acc.at[dst_v.at[0]], sem_sa).wait()
        pltpu.make_async_copy(rows_b, acc.at[dst_v.at[0]], sem_sb).wait()

      plsc.subcore_barrier()
      pltpu.sync_copy(acc.at[pl.ds(sid * OUT_STEP, OUT_ROWS)],
                      out_hbm.at[core, c, pl.ds(sid * OUT_STEP, OUT_ROWS)])
      plsc.subcore_barrier()

  return scatter


def _k1_body(x_ref, w_ref, d0_ref, d1_ref, out_ref, dinv_ref):
  dinv = lax.rsqrt(d0_ref[0, 0] + d1_ref[0, 0] + 1.0)
  out_ref[...] = jnp.dot(x_ref[...], w_ref[...],
                         preferred_element_type=jnp.float32) * dinv
  dinv_ref[...] = dinv


def _k1_call(x, W1, degt):
  co = D_H // 128
  return pl.pallas_call(
      _k1_body,
      grid=(co, GM),
      in_specs=[
          pl.BlockSpec((BM, D_IN), lambda c, i: (i, 0)),
          pl.BlockSpec((D_IN, 128), lambda c, i: (0, c)),
          pl.BlockSpec((1, 1, BM, 128), lambda c, i: (0, 0, i, 0)),
          pl.BlockSpec((1, 1, BM, 128), lambda c, i: (1, 0, i, 0)),
      ],
      out_specs=[
          pl.BlockSpec((BM, 128), lambda c, i: (c * GM + i, 0)),
          pl.BlockSpec((BM, 128), lambda c, i: (i, 0)),
      ],
      out_shape=[
          jax.ShapeDtypeStruct((co * N, 128), jnp.float32),
          jax.ShapeDtypeStruct((N, 128), jnp.float32),
      ],
      compiler_params=pltpu.CompilerParams(
          dimension_semantics=("parallel", "parallel")),
  )(x, W1, degt, degt)


def _kmid_body(t0_ref, t1_ref, g_ref, dinv_ref, b_ref, w_ref, out_ref, *,
               n_ci, scale_out):
  ci = pl.program_id(2)
  act = (t0_ref[0, 0] + t1_ref[0, 0] + g_ref[...]) * dinv_ref[...] + b_ref[0]
  act = jnp.maximum(act, 0.0)
  part = jnp.dot(act, w_ref[...], preferred_element_type=jnp.float32)

  @pl.when(ci == 0)
  def _():
    out_ref[...] = part

  @pl.when(ci > 0)
  def _():
    out_ref[...] += part

  if scale_out:
    @pl.when(ci == n_ci - 1)
    def _():
      out_ref[...] *= dinv_ref[...]


def _kmid_call(t, g, dinvp, b, W, d_out, scale_out=True):
  n_ci = D_H // 128
  n_co = d_out // 128
  body = functools.partial(_kmid_body, n_ci=n_ci, scale_out=scale_out)
  return pl.pallas_call(
      body,
      grid=(n_co, GM, n_ci),
      in_specs=[
          pl.BlockSpec((1, 1, BM, 128), lambda co, i, ci: (0, ci, i, 0)),
          pl.BlockSpec((1, 1, BM, 128), lambda co, i, ci: (1, ci, i, 0)),
          pl.BlockSpec((BM, 128), lambda co, i, ci: (ci * GM + i, 0)),
          pl.BlockSpec((BM, 128), lambda co, i, ci: (i, 0)),
          pl.BlockSpec((1, 1, 128), lambda co, i, ci: (ci, 0, 0)),
          pl.BlockSpec((128, 128), lambda co, i, ci: (ci, co)),
      ],
      out_specs=pl.BlockSpec((BM, 128), lambda co, i, ci: (co * GM + i, 0)),
      out_shape=jax.ShapeDtypeStruct((n_co * N, 128), jnp.float32),
      compiler_params=pltpu.CompilerParams(
          dimension_semantics=("parallel", "parallel", "arbitrary")),
  )(t, t, g, dinvp, b, W)


def _kfin_body(t0_ref, t1_ref, g_ref, dinv_ref, b_ref, out_ref):
  z = (t0_ref[0, 0] + t1_ref[0, 0] + g_ref[...]) * dinv_ref[...] + b_ref[0]
  m = jnp.max(z, axis=1, keepdims=True)
  e = jnp.exp(z - m)
  s = jnp.sum(e, axis=1, keepdims=True)
  out_ref[...] = z - m - jnp.log(s)


def _kfin_call(t, g, dinvp, bo):
  return pl.pallas_call(
      _kfin_body,
      grid=(GM,),
      in_specs=[
          pl.BlockSpec((1, 1, BM, 128), lambda i: (0, 0, i, 0)),
          pl.BlockSpec((1, 1, BM, 128), lambda i: (1, 0, i, 0)),
          pl.BlockSpec((BM, 128), lambda i: (i, 0)),
          pl.BlockSpec((BM, 128), lambda i: (i, 0)),
          pl.BlockSpec((1, 1, 128), lambda i: (0, 0, 0)),
      ],
      out_specs=pl.BlockSpec((BM, 128), lambda i: (i, 0)),
      out_shape=jax.ShapeDtypeStruct((N, D_OUT), jnp.float32),
  )(t, t, g, dinvp, bo)


def kernel(x, edge_index, W1, b1, W2, b2, W3, b3, Wo, bo):
  src = edge_index[0].astype(jnp.int32)
  dst = edge_index[1].astype(jnp.int32)

  npad = E_PAD - E
  pad_src = (jnp.arange(npad, dtype=jnp.int32) * 16) % N
  pad_dst = N + (jnp.arange(npad, dtype=jnp.int32) % (NPAD - N - 8))
  src_p = jnp.concatenate([src, pad_src])
  dst_p = jnp.concatenate([dst, pad_dst])
  dst_t = dst_p.reshape(2, 16, NB, K)
  src_t = src_p.reshape(2, 16, NB, K)

  zeros_st = jnp.zeros((ZR, 128), jnp.float32)
  ones_st = jnp.ones((K, 128), jnp.float32)
  dummy_g = jnp.zeros((8, 128), jnp.float32)

  degt = _scatter_sc(1, True)(dummy_g, src_t, dst_t, zeros_st, ones_st)

  scat4 = _scatter_sc(4, False)
  scat1 = _scatter_sc(1, False)

  b1r = b1.reshape(4, 1, 128)
  b2r = b2.reshape(4, 1, 128)
  b3r = b3.reshape(4, 1, 128)
  bor = bo.reshape(1, 1, 128)

  g1, dinvp = _k1_call(x, W1, degt)
  t1 = scat4(g1, src_t, dst_t, zeros_st, ones_st)
  g2 = _kmid_call(t1, g1, dinvp, b1r, W2, D_H)
  t2 = scat4(g2, src_t, dst_t, zeros_st, ones_st)
  g3 = _kmid_call(t2, g2, dinvp, b2r, W3, D_H)
  t3 = scat4(g3, src_t, dst_t, zeros_st, ones_st)
  g4 = _kmid_call(t3, g3, dinvp, b3r, Wo, D_OUT)
  t4 = scat1(g4, src_t, dst_t, zeros_st, ones_st)
  return _kfin_call(t4, g4, dinvp, bor)


# speedup vs baseline: 13.7219x; 1.2053x over previous
import functools

import jax
import jax.numpy as jnp
from jax import lax
from jax.experimental import pallas as pl
from jax.experimental.pallas import tpu as pltpu
from jax.experimental.pallas import tpu_sc as plsc

N = 10000
E = 160000
D_IN = 256
D_H = 512
D_OUT = 128

K = 128
NB = 40
E_PAD = 2 * 16 * NB * K
PAD_DST = 10008
NPAD = 10240
ZR = 40
OUT_STEP = 624
OUT_ROWS = 640

BM = 2000
GM = N // BM


def _scatter_sc(C, use_ones):
  mesh = plsc.VectorSubcoreMesh(core_axis_name="c", subcore_axis_name="s")

  @functools.partial(
      pl.kernel,
      mesh=mesh,
      out_type=jax.ShapeDtypeStruct((2, C, N, 128), jnp.float32),
      scratch_types=[
          pltpu.VMEM_SHARED((NPAD, 128), jnp.float32),
          pltpu.VMEM((ZR, 128), jnp.float32),
          pltpu.VMEM((NB, K), jnp.int32),
          pltpu.VMEM((NB, K), jnp.int32),
          pltpu.VMEM((K, 128), jnp.float32),
          pltpu.VMEM((K, 128), jnp.float32),
          pltpu.SemaphoreType.DMA,
          pltpu.SemaphoreType.DMA,
          pltpu.SemaphoreType.DMA,
          pltpu.SemaphoreType.DMA,
      ],
  )
  def scatter(*args):
    g_planes = args[:C]
    src_hbm, dst_hbm, zeros_hbm, ones_hbm, out_hbm = args[C:C + 5]
    (acc, zbuf, src_v, dst_v, rows_a, rows_b,
     sem_a, sem_b, sem_sa, sem_sb) = args[C + 5:]
    core = lax.axis_index("c")
    sid = lax.axis_index("s")

    pltpu.sync_copy(dst_hbm.at[core, sid], dst_v)
    pltpu.sync_copy(zeros_hbm, zbuf)
    if use_ones:
      pltpu.sync_copy(ones_hbm, rows_a)
    else:
      pltpu.sync_copy(src_hbm.at[core, sid], src_v)

    for c in range(C):
      g_hbm = g_planes[c]
      def zinit(q, _):
        pltpu.sync_copy(zbuf, acc.at[pl.ds(sid * 16 * ZR + q * ZR, ZR)])
        return 0
      lax.fori_loop(0, 16, zinit, 0)
      plsc.subcore_barrier()

      if use_ones:
        def body1(j, _):
          pltpu.sync_copy(rows_a, acc.at[dst_v.at[j]], add=True)
          return 0
        lax.fori_loop(0, NB, body1, 0)
      else:
        pltpu.async_copy(g_hbm.at[src_v.at[0]], rows_a, sem_a)

        def body2(i, _):
          ja = 2 * i

          @pl.when(i > 0)
          def _():
            pltpu.make_async_copy(rows_b, acc.at[dst_v.at[0]], sem_sb).wait()

          pltpu.async_copy(g_hbm.at[src_v.at[ja + 1]], rows_b, sem_b)
          pltpu.make_async_copy(g_hbm.at[src_v.at[ja]], rows_a, sem_a).wait()
          pltpu.async_copy(rows_a, acc.at[dst_v.at[ja]], sem_sa, add=True)

          @pl.when(ja + 2 < NB)
          def _():
            pltpu.make_async_copy(rows_a, acc.at[dst_v.at[0]], sem_sa).wait()
            pltpu.async_copy(g_hbm.at[src_v.at[ja + 2]], rows_a, sem_a)

          pltpu.make_async_copy(g_hbm.at[src_v.at[ja + 1]], rows_b, sem_b).wait()
          pltpu.async_copy(rows_b, acc.at[dst_v.at[ja + 1]], sem_sb, add=True)
          return 0

        lax.fori_loop(0, NB // 2, body2, 0)
        pltpu.make_async_copy(rows_a, acc.at[dst_v.at[0]], sem_sa).wait()
        pltpu.make_async_copy(rows_b, acc.at[dst_v.at[0]], sem_sb).wait()

      plsc.subcore_barrier()
      pltpu.sync_copy(acc.at[pl.ds(sid * OUT_STEP, OUT_ROWS)],
                      out_hbm.at[core, c, pl.ds(sid * OUT_STEP, OUT_ROWS)])
      plsc.subcore_barrier()

  return scatter


def _k1_body(x_ref, w_ref, d0_ref, d1_ref, *out_refs):
  dinv = lax.rsqrt(d0_ref[0, 0] + d1_ref[0, 0] + 1.0)
  prod = jnp.dot(x_ref[...], w_ref[...],
                 preferred_element_type=jnp.float32) * dinv[:, :1]
  for c in range(D_H // 128):
    out_refs[c][...] = prod[:, c * 128:(c + 1) * 128]
  out_refs[-1][...] = dinv


def _k1_call(x, W1, degt):
  co = D_H // 128
  return pl.pallas_call(
      _k1_body,
      grid=(GM,),
      in_specs=[
          pl.BlockSpec((BM, D_IN), lambda i: (i, 0)),
          pl.BlockSpec((D_IN, D_H), lambda i: (0, 0)),
          pl.BlockSpec((1, 1, BM, 128), lambda i: (0, 0, i, 0)),
          pl.BlockSpec((1, 1, BM, 128), lambda i: (1, 0, i, 0)),
      ],
      out_specs=[pl.BlockSpec((BM, 128), lambda i: (i, 0))] * (co + 1),
      out_shape=[jax.ShapeDtypeStruct((N, 128), jnp.float32)] * (co + 1),
      compiler_params=pltpu.CompilerParams(
          dimension_semantics=("parallel",)),
  )(x, W1, degt, degt)


def _kmid_body(t0_ref, t1_ref, g0, g1, g2, g3, dinv_ref, b_ref, w_ref,
               *out_refs, n_co, scale_out):
  gcat = jnp.concatenate([g0[...], g1[...], g2[...], g3[...]], axis=1)
  tcat0 = jnp.concatenate([t0_ref[0, c] for c in range(4)], axis=1)
  tcat1 = jnp.concatenate([t1_ref[0, c] for c in range(4)], axis=1)
  dinv = dinv_ref[:, :1]
  act = (tcat0 + tcat1 + gcat) * dinv + b_ref[...]
  act = jnp.maximum(act, 0.0)
  prod = jnp.dot(act, w_ref[...], preferred_element_type=jnp.float32)
  if scale_out:
    prod = prod * dinv
  for c in range(n_co):
    out_refs[c][...] = prod[:, c * 128:(c + 1) * 128]


def _kmid_call(t, gp, dinvp, b, W, d_out, scale_out=True):
  n_co = d_out // 128
  body = functools.partial(_kmid_body, n_co=n_co, scale_out=scale_out)
  outs = pl.pallas_call(
      body,
      grid=(GM,),
      in_specs=[
          pl.BlockSpec((1, 4, BM, 128), lambda i: (0, 0, i, 0)),
          pl.BlockSpec((1, 4, BM, 128), lambda i: (1, 0, i, 0)),
          pl.BlockSpec((BM, 128), lambda i: (i, 0)),
          pl.BlockSpec((BM, 128), lambda i: (i, 0)),
          pl.BlockSpec((BM, 128), lambda i: (i, 0)),
          pl.BlockSpec((BM, 128), lambda i: (i, 0)),
          pl.BlockSpec((BM, 128), lambda i: (i, 0)),
          pl.BlockSpec((1, D_H), lambda i: (0, 0)),
          pl.BlockSpec((D_H, d_out), lambda i: (0, 0)),
      ],
      out_specs=[pl.BlockSpec((BM, 128), lambda i: (i, 0))] * n_co,
      out_shape=[jax.ShapeDtypeStruct((N, 128), jnp.float32)] * n_co,
      compiler_params=pltpu.CompilerParams(
          dimension_semantics=("parallel",)),
  )(t, t, gp[0], gp[1], gp[2], gp[3], dinvp, b, W)
  return outs


def _kfin_body(t0_ref, t1_ref, g_ref, dinv_ref, b_ref, out_ref):
  z = ((t0_ref[0, 0] + t1_ref[0, 0] + g_ref[...]) * dinv_ref[:, :1]
       + b_ref[...])
  m = jnp.max(z, axis=1, keepdims=True)
  e = jnp.exp(z - m)
  s = jnp.sum(e, axis=1, keepdims=True)
  out_ref[...] = z - m - jnp.log(s)


def _kfin_call(t, g, dinvp, bo):
  return pl.pallas_call(
      _kfin_body,
      grid=(GM,),
      in_specs=[
          pl.BlockSpec((1, 1, BM, 128), lambda i: (0, 0, i, 0)),
          pl.BlockSpec((1, 1, BM, 128), lambda i: (1, 0, i, 0)),
          pl.BlockSpec((BM, 128), lambda i: (i, 0)),
          pl.BlockSpec((BM, 128), lambda i: (i, 0)),
          pl.BlockSpec((1, 128), lambda i: (0, 0)),
      ],
      out_specs=pl.BlockSpec((BM, 128), lambda i: (i, 0)),
      out_shape=jax.ShapeDtypeStruct((N, D_OUT), jnp.float32),
  )(t, t, g, dinvp, bo)


def kernel(x, edge_index, W1, b1, W2, b2, W3, b3, Wo, bo):
  src = edge_index[0].astype(jnp.int32)
  dst = edge_index[1].astype(jnp.int32)

  npad = E_PAD - E
  pad_src = (jnp.arange(npad, dtype=jnp.int32) * 16) % N
  pad_dst = N + (jnp.arange(npad, dtype=jnp.int32) % (NPAD - N - 8))
  src_p = jnp.concatenate([src, pad_src])
  dst_p = jnp.concatenate([dst, pad_dst])
  dst_t = dst_p.reshape(2, 16, NB, K)
  src_t = src_p.reshape(2, 16, NB, K)

  zeros_st = jnp.zeros((ZR, 128), jnp.float32)
  ones_st = jnp.ones((K, 128), jnp.float32)
  dummy_g = jnp.zeros((8, 128), jnp.float32)

  degt = _scatter_sc(1, True)(dummy_g, src_t, dst_t, zeros_st, ones_st)

  scat4 = _scatter_sc(4, False)
  scat1 = _scatter_sc(1, False)

  b1r = b1.reshape(1, D_H)
  b2r = b2.reshape(1, D_H)
  b3r = b3.reshape(1, D_H)
  bor = bo.reshape(1, D_OUT)

  *g1, dinvp = _k1_call(x, W1, degt)
  t1 = scat4(*g1, src_t, dst_t, zeros_st, ones_st)
  g2 = _kmid_call(t1, g1, dinvp, b1r, W2, D_H)
  t2 = scat4(*g2, src_t, dst_t, zeros_st, ones_st)
  g3 = _kmid_call(t2, g2, dinvp, b2r, W3, D_H)
  t3 = scat4(*g3, src_t, dst_t, zeros_st, ones_st)
  g4 = _kmid_call(t3, g3, dinvp, b3r, Wo, D_OUT)
  t4 = scat1(g4[0], src_t, dst_t, zeros_st, ones_st)
  return _kfin_call(t4, g4[0], dinvp, bor)
